# baseline (speedup 1.0000x reference)
import functools

import jax
import jax.numpy as jnp
from jax import lax
from jax.experimental import pallas as pl
from jax.experimental.pallas import tpu as pltpu
from jax.experimental.pallas import tpu_sc as plsc

K = 32
B = 16384
NC, NS, L = 2, 16, 16
NW = NC * NS
BPW = B // NW
NCHUNK = BPW // L
GCH = 128
NG = BPW // GCH


def _sc_gather_dots(u, i, j, gamma_u, gamma_i, beta_u, beta_i):
  mesh = plsc.VectorSubcoreMesh(core_axis_name="c", subcore_axis_name="s")

  @functools.partial(
      pl.kernel,
      out_type=[jax.ShapeDtypeStruct((B,), jnp.float32),
                jax.ShapeDtypeStruct((B,), jnp.float32)],
      mesh=mesh,
      compiler_params=pltpu.CompilerParams(
          needs_layout_passes=False, use_tc_tiling_on_sc=False),
      scratch_types=[
          pltpu.VMEM((BPW,), jnp.int32),
          pltpu.VMEM((BPW,), jnp.int32),
          pltpu.VMEM((BPW,), jnp.int32),
          pltpu.VMEM((BPW, K), jnp.float32),
          pltpu.VMEM((BPW, K), jnp.float32),
          pltpu.VMEM((BPW, K), jnp.float32),
          pltpu.VMEM((BPW,), jnp.float32),
          pltpu.VMEM((BPW,), jnp.float32),
          pltpu.VMEM((BPW,), jnp.float32),
          pltpu.VMEM((BPW,), jnp.float32),
          pltpu.VMEM((BPW,), jnp.float32),
          pltpu.SemaphoreType.DMA,
      ],
  )
  def sc_k(u_h, i_h, j_h, gu_h, gi_h, bu_h, bi_h, xui_h, xuj_h,
           iu_v, ii_v, ij_v, ru_v, ri_v, rj_v, bu_v, bi_v, bj_v,
           xui_v, xuj_v, sem):
    wid = lax.axis_index("s") * NC + lax.axis_index("c")
    base = wid * BPW
    pltpu.sync_copy(u_h.at[pl.ds(base, BPW)], iu_v)
    pltpu.sync_copy(i_h.at[pl.ds(base, BPW)], ii_v)
    pltpu.sync_copy(j_h.at[pl.ds(base, BPW)], ij_v)

    copies = []
    for t in range(NG):
      sl = pl.ds(t * GCH, GCH)
      copies.append(pltpu.async_copy(gu_h.at[iu_v.at[sl]], ru_v.at[sl], sem))
      copies.append(pltpu.async_copy(gi_h.at[ii_v.at[sl]], ri_v.at[sl], sem))
      copies.append(pltpu.async_copy(gi_h.at[ij_v.at[sl]], rj_v.at[sl], sem))
      copies.append(pltpu.async_copy(bu_h.at[iu_v.at[sl]], bu_v.at[sl], sem))
      copies.append(pltpu.async_copy(bi_h.at[ii_v.at[sl]], bi_v.at[sl], sem))
      copies.append(pltpu.async_copy(bi_h.at[ij_v.at[sl]], bj_v.at[sl], sem))
    for cp in copies:
      cp.wait()

    lane = lax.iota(jnp.int32, L)

    def chunk(c, carry):
      off = pl.multiple_of(c * L, L)
      b_u = bu_v[pl.ds(off, L)]
      acc_ui = b_u + bi_v[pl.ds(off, L)]
      acc_uj = b_u + bj_v[pl.ds(off, L)]
      for m in range(L):
        b = c * L + m
        pu0 = ru_v[b, pl.ds(0, L)]
        pu1 = ru_v[b, pl.ds(L, L)]
        pi0 = ri_v[b, pl.ds(0, L)]
        pi1 = ri_v[b, pl.ds(L, L)]
        pj0 = rj_v[b, pl.ds(0, L)]
        pj1 = rj_v[b, pl.ds(L, L)]
        dui = pu0 * pi0 + pu1 * pi1
        duj = pu0 * pj0 + pu1 * pj1
        msk = lane == m
        acc_ui = jnp.where(msk, acc_ui + jnp.sum(dui), acc_ui)
        acc_uj = jnp.where(msk, acc_uj + jnp.sum(duj), acc_uj)
      xui_v[pl.ds(off, L)] = acc_ui
      xuj_v[pl.ds(off, L)] = acc_uj
      return carry

    lax.fori_loop(0, NCHUNK, chunk, 0)
    pltpu.sync_copy(xui_v, xui_h.at[pl.ds(base, BPW)])
    pltpu.sync_copy(xuj_v, xuj_h.at[pl.ds(base, BPW)])

  return sc_k(u, i, j, gamma_u, gamma_i, beta_u, beta_i)


def _loss_body(a_ref, b_ref, o_ref):
  z = a_ref[...] - b_ref[...]
  ls = jnp.minimum(z, 0.0) - jnp.log1p(jnp.exp(-jnp.abs(z)))
  o_ref[0, 0] = -jnp.sum(ls) / jnp.float32(B)


def _loss(xui, xuj):
  out = pl.pallas_call(
      _loss_body,
      out_shape=jax.ShapeDtypeStruct((1, 1), jnp.float32),
      out_specs=pl.BlockSpec(memory_space=pltpu.SMEM),
  )(xui.reshape(128, 128), xuj.reshape(128, 128))
  return out[0, 0]


@jax.jit
def kernel(u, i, j, gamma_u, gamma_i, beta_u, beta_i):
  u = u.astype(jnp.int32)
  i = i.astype(jnp.int32)
  j = j.astype(jnp.int32)
  xui, xuj = _sc_gather_dots(u, i, j, gamma_u, gamma_i,
                             beta_u.reshape(-1), beta_i.reshape(-1))
  loss = _loss(xui, xuj)
  return (xui, xuj, loss)
